# baseline (speedup 1.0000x reference)
import jax
import jax.numpy as jnp
from jax.experimental import pallas as pl
from jax.experimental.pallas import tpu as pltpu

_E, _K, _H, _M, _SH = 64, 6, 128, 80, 160
_T = 1024
_EH = 2
_EW = _E // _EH * _M


def _moe_block_kernel(x_ref, r_ref, wg_ref, wgt_ref, wdt_ref,
                      rmap_ref, wsg_ref, wsd_ref, y_ref, scores_ref,
                      wts_ref):
    e = pl.program_id(1)
    x = x_ref[...]
    xb = x.astype(jnp.bfloat16)

    @pl.when(e == 0)
    def _router_and_shared():
        r = r_ref[...]
        logits = jnp.dot(r, wg_ref[...],
                         preferred_element_type=jnp.float32)
        mx = jnp.max(logits, axis=1, keepdims=True)
        ex = jnp.exp(logits - mx)
        scores = ex / jnp.sum(ex, axis=1, keepdims=True)
        scores_ref[...] = scores

        iota = jax.lax.broadcasted_iota(jnp.int32, scores.shape, 1)
        sbits = jax.lax.bitcast_convert_type(scores, jnp.int32)
        key = jax.lax.bitwise_or(
            jax.lax.bitwise_and(sbits, ~jnp.int32(_E - 1)),
            (_E - 1) - iota)
        sel = jnp.zeros(scores.shape, jnp.bool_)
        for _ in range(_K):
            m = jnp.max(key, axis=1, keepdims=True)
            pick = key == m
            sel = jnp.logical_or(sel, pick)
            key = jnp.where(pick, jnp.int32(-1), key)
        wts = jnp.where(sel, scores, 0.0)
        wts_ref[...] = wts / (jnp.sum(wts, axis=1, keepdims=True) + 1e-20)

        sh = jnp.dot(xb, wsg_ref[...],
                     preferred_element_type=jnp.float32).astype(jnp.bfloat16)
        sg = sh[:, :_SH]
        su = sh[:, _SH:]
        sact = (sg + sg * jnp.tanh(sg)) * su
        y_ref[...] = x + jnp.dot(sact, wsd_ref[...],
                                 preferred_element_type=jnp.float32)

    h = jnp.dot(xb, wgt_ref[0],
                preferred_element_type=jnp.float32).astype(jnp.bfloat16)
    h1 = h[:, :_EW]
    h2 = h[:, _EW:]
    act = (h1 + h1 * jnp.tanh(h1)) * h2
    wwide = jnp.dot(wts_ref[...].astype(jnp.bfloat16), rmap_ref[0],
                    preferred_element_type=jnp.float32).astype(jnp.bfloat16)
    scaled = act * wwide
    y_ref[...] += jnp.dot(scaled, wdt_ref[0],
                          preferred_element_type=jnp.float32)


def kernel(hidden_states, tgt_route, W_gate, Wg, Wu, Wd, Ws_g, Ws_u, Ws_d):
    B, S, H = hidden_states.shape
    N = B * S
    x = hidden_states.reshape(N, H)
    r = tgt_route.reshape(N, H)

    wgT = W_gate.T
    wgtT = (0.5 * Wg.transpose(2, 0, 1).reshape(H, _E * _M)
            ).astype(jnp.bfloat16)
    wutT = Wu.transpose(2, 0, 1).reshape(H, _E * _M).astype(jnp.bfloat16)
    wgu3 = jnp.stack(
        [jnp.concatenate([wgtT[:, i * _EW:(i + 1) * _EW],
                          wutT[:, i * _EW:(i + 1) * _EW]], axis=1)
         for i in range(_EH)])
    wdt3 = Wd.transpose(0, 2, 1).reshape(_E * _M, H).astype(
        jnp.bfloat16).reshape(_EH, _EW, H)
    lane_e = jnp.arange(_EW)[None, :] // _M
    rmap3 = jnp.stack(
        [(jnp.arange(_E)[:, None] == (lane_e + i * (_E // _EH))
          ).astype(jnp.bfloat16) for i in range(_EH)])
    wsguT = jnp.concatenate([0.5 * Ws_g.T, Ws_u.T],
                            axis=1).astype(jnp.bfloat16)
    wsdT = Ws_d.T.astype(jnp.bfloat16)

    grid = (N // _T, _EH)
    tok = lambda i, e: (i, 0)
    full = lambda i, e: (0, 0)
    eh3 = lambda i, e: (e, 0, 0)
    y, scores = pl.pallas_call(
        _moe_block_kernel,
        grid=grid,
        in_specs=[
            pl.BlockSpec((_T, H), tok),
            pl.BlockSpec((_T, H), tok),
            pl.BlockSpec((H, _E), full),
            pl.BlockSpec((1, H, 2 * _EW), eh3),
            pl.BlockSpec((1, _EW, H), eh3),
            pl.BlockSpec((1, _E, _EW), eh3),
            pl.BlockSpec((H, 2 * _SH), full),
            pl.BlockSpec((_SH, H), full),
        ],
        out_specs=[
            pl.BlockSpec((_T, H), tok),
            pl.BlockSpec((_T, _E), tok),
        ],
        out_shape=[
            jax.ShapeDtypeStruct((N, H), jnp.float32),
            jax.ShapeDtypeStruct((N, _E), jnp.float32),
        ],
        scratch_shapes=[pltpu.VMEM((_T, _E), jnp.float32)],
        compiler_params=pltpu.CompilerParams(
            dimension_semantics=("parallel", "arbitrary")),
    )(x, r, wgT, wgu3, wdt3, rmap3, wsguT, wsdT)
    return y.reshape(B, S, H), scores


# speedup vs baseline: 1.1051x; 1.1051x over previous
import jax
import jax.numpy as jnp
from jax.experimental import pallas as pl
from jax.experimental.pallas import tpu as pltpu

_E, _K, _H, _M, _SH = 64, 6, 128, 80, 160
_T = 1024


def _moe_block_kernel(x_ref, r_ref, wg_ref, wgt_ref, wdt_ref,
                      rmap_ref, wsg_ref, wsd_ref, y_ref, scores_ref):
    x = x_ref[...]
    r = r_ref[...]

    logits = jnp.dot(r, wg_ref[...], preferred_element_type=jnp.float32)
    mx = jnp.max(logits, axis=1, keepdims=True)
    ex = jnp.exp(logits - mx)
    scores = ex / jnp.sum(ex, axis=1, keepdims=True)
    scores_ref[...] = scores

    iota = jax.lax.broadcasted_iota(jnp.int32, scores.shape, 1)
    sbits = jax.lax.bitcast_convert_type(scores, jnp.int32)
    key = jax.lax.bitwise_or(jax.lax.bitwise_and(sbits, ~jnp.int32(_E - 1)),
                             (_E - 1) - iota)
    sel = jnp.zeros(scores.shape, jnp.bool_)
    for _ in range(_K):
        m = jnp.max(key, axis=1, keepdims=True)
        pick = key == m
        sel = jnp.logical_or(sel, pick)
        key = jnp.where(pick, jnp.int32(-1), key)
    wts = jnp.where(sel, scores, 0.0)
    wts = wts / (jnp.sum(wts, axis=1, keepdims=True) + 1e-20)

    xb = x.astype(jnp.bfloat16)
    h = jnp.dot(xb, wgt_ref[...],
                preferred_element_type=jnp.float32).astype(jnp.bfloat16)
    h1 = h[:, :_E * _M]
    h2 = h[:, _E * _M:]
    act = (h1 + h1 * jnp.tanh(h1)) * h2
    wwide = jnp.dot(wts.astype(jnp.bfloat16), rmap_ref[...],
                    preferred_element_type=jnp.float32).astype(jnp.bfloat16)
    scaled = act * wwide
    y = jnp.dot(scaled, wdt_ref[...], preferred_element_type=jnp.float32)

    sh = jnp.dot(xb, wsg_ref[...],
                 preferred_element_type=jnp.float32).astype(jnp.bfloat16)
    sg = sh[:, :_SH]
    su = sh[:, _SH:]
    sact = (sg + sg * jnp.tanh(sg)) * su
    y = y + jnp.dot(sact, wsd_ref[...], preferred_element_type=jnp.float32)

    y_ref[...] = y + x


def kernel(hidden_states, tgt_route, W_gate, Wg, Wu, Wd, Ws_g, Ws_u, Ws_d):
    B, S, H = hidden_states.shape
    N = B * S
    x = hidden_states.reshape(N, H)
    r = tgt_route.reshape(N, H)

    wgT = W_gate.T
    wgtT = (0.5 * Wg.transpose(2, 0, 1).reshape(H, _E * _M)
            ).astype(jnp.bfloat16)
    wutT = Wu.transpose(2, 0, 1).reshape(H, _E * _M).astype(jnp.bfloat16)
    wguT = jnp.concatenate([wgtT, wutT], axis=1)
    wdtT = Wd.transpose(0, 2, 1).reshape(_E * _M, H).astype(jnp.bfloat16)
    rmap = (jnp.arange(_E)[:, None] == (jnp.arange(_E * _M)[None, :] // _M)
            ).astype(jnp.bfloat16)
    wsguT = jnp.concatenate([0.5 * Ws_g.T, Ws_u.T],
                            axis=1).astype(jnp.bfloat16)
    wsdT = Ws_d.T.astype(jnp.bfloat16)

    grid = (N // _T,)
    tok = lambda i: (i, 0)
    full = lambda i: (0, 0)
    y, scores = pl.pallas_call(
        _moe_block_kernel,
        grid=grid,
        in_specs=[
            pl.BlockSpec((_T, H), tok),
            pl.BlockSpec((_T, H), tok),
            pl.BlockSpec((H, _E), full),
            pl.BlockSpec((H, 2 * _E * _M), full),
            pl.BlockSpec((_E * _M, H), full),
            pl.BlockSpec((_E, _E * _M), full),
            pl.BlockSpec((H, 2 * _SH), full),
            pl.BlockSpec((_SH, H), full),
        ],
        out_specs=[
            pl.BlockSpec((_T, H), tok),
            pl.BlockSpec((_T, _E), tok),
        ],
        out_shape=[
            jax.ShapeDtypeStruct((N, H), jnp.float32),
            jax.ShapeDtypeStruct((N, _E), jnp.float32),
        ],
        compiler_params=pltpu.CompilerParams(
            dimension_semantics=("parallel",)),
    )(x, r, wgT, wguT, wdtT, rmap, wsguT, wsdT)
    return y.reshape(B, S, H), scores
